# baseline (speedup 1.0000x reference)
import jax
import jax.numpy as jnp
from jax.experimental import pallas as pl


def _encoder_kernel(x_ref, wc_ref, bc_ref, w1_ref, b1_ref, w2_ref, b2_ref,
                    out_ref):
    xblk = x_ref[...]
    wct = jnp.transpose(wc_ref[...])
    s_t = jnp.dot(wct, xblk, preferred_element_type=jnp.float32)
    s_t = s_t + bc_ref[...]
    w1 = w1_ref[...]
    w1t_eff = jnp.transpose(w1[:8, :] + w1[8:, :])
    h_t = jnp.dot(w1t_eff, s_t, preferred_element_type=jnp.float32)
    h_t = h_t + b1_ref[...]
    h_t = jnp.where(h_t >= 0, h_t, 0.01 * h_t)
    o_t = jnp.dot(jnp.transpose(w2_ref[...]), h_t,
                  preferred_element_type=jnp.float32)
    o_t = o_t + b2_ref[...]
    out_ref[...] = jnp.broadcast_to(o_t, out_ref.shape)


def kernel(x, edge_index, edge_attr, W_conv, b_conv, W1, b1, W2, b2):
    del edge_index, edge_attr
    B, L, N = x.shape
    BL = B * L
    x2d = x.reshape(BL, N)
    NB = 2048
    grid = (N // NB,)

    full = lambda shape: pl.BlockSpec(shape, lambda i: (0, 0))
    out2d = pl.pallas_call(
        _encoder_kernel,
        grid=grid,
        in_specs=[
            pl.BlockSpec((BL, NB), lambda i: (0, i)),
            full((BL, 8)),
            full((8, 1)),
            full((16, 32)),
            full((32, 1)),
            full((32, 1)),
            full((1, 1)),
        ],
        out_specs=pl.BlockSpec((BL, NB), lambda i: (0, i)),
        out_shape=jax.ShapeDtypeStruct((BL, N), jnp.float32),
    )(
        x2d,
        W_conv,
        b_conv.reshape(8, 1),
        W1,
        b1.reshape(32, 1),
        W2,
        b2.reshape(1, 1),
    )
    return out2d.reshape(B, L, N)


# speedup vs baseline: 1.5800x; 1.5800x over previous
import jax
import jax.numpy as jnp
from jax.experimental import pallas as pl


def _encoder_kernel(x_ref, wct_ref, bc_ref, w1_ref, b1_ref, w2_ref, b2_ref,
                    out_ref):
    xblk = x_ref[...]
    wct = wct_ref[...]
    s_t = jnp.dot(wct, xblk, preferred_element_type=jnp.float32)
    s_t = s_t + jnp.transpose(bc_ref[...])
    w1 = w1_ref[...]
    w1t_eff = jnp.transpose(w1[:8, :] + w1[8:, :])
    h_t = jnp.dot(w1t_eff, s_t, preferred_element_type=jnp.float32)
    h_t = h_t + jnp.transpose(b1_ref[...])
    h_t = jnp.where(h_t >= 0, h_t, 0.01 * h_t)
    o_t = jnp.dot(jnp.transpose(w2_ref[...]), h_t,
                  preferred_element_type=jnp.float32)
    o_t = o_t + b2_ref[...]
    out_ref[...] = jnp.broadcast_to(o_t, out_ref.shape)


def kernel(x, edge_index, edge_attr, W_conv, b_conv, W1, b1, W2, b2):
    del edge_index, edge_attr
    B, L, N = x.shape
    BL = B * L
    x2d = x.reshape(BL, N)
    NB = 2048
    grid = (N // NB,)

    full = lambda shape: pl.BlockSpec(shape, lambda i: (0, 0))
    out2d = pl.pallas_call(
        _encoder_kernel,
        grid=grid,
        in_specs=[
            pl.BlockSpec((BL, NB), lambda i: (0, i)),
            full((8, BL)),
            full((1, 8)),
            full((16, 32)),
            full((1, 32)),
            full((32, 1)),
            full((1, 1)),
        ],
        out_specs=pl.BlockSpec((BL, NB), lambda i: (0, i)),
        out_shape=jax.ShapeDtypeStruct((BL, N), jnp.float32),
    )(
        x2d,
        W_conv.T,
        b_conv.reshape(1, 8),
        W1,
        b1.reshape(1, 32),
        W2,
        b2.reshape(1, 1),
    )
    return out2d.reshape(B, L, N)
